# baseline (speedup 1.0000x reference)
import functools

import jax
import jax.numpy as jnp
from jax import lax
from jax.experimental import pallas as pl
from jax.experimental.pallas import tpu as pltpu
from jax.experimental.pallas import tpu_sc as plsc

VOCAB = 7
EMBED = 4
LANES = 16

NJ = 200
NI = 16384
JT, JS = 25, 8
IT, IL = 128, 128
WA, WB = 4, 8
JPW = NJ // WA
TPW = IT // WB


@functools.lru_cache(maxsize=None)
def _make_sc_kernel():
    mesh = plsc.VectorSubcoreMesh(core_axis_name="c", subcore_axis_name="s")
    info = plsc.get_sparse_core_info()
    nc = info.num_cores

    @functools.partial(
        pl.kernel,
        mesh=mesh,
        out_type=jax.ShapeDtypeStruct((NJ, EMBED * IT, IL), jnp.float32),
        scratch_types=[
            pltpu.VMEM((32,), jnp.float32),
            pltpu.VMEM((2, TPW, IL), jnp.int32),
            pltpu.VMEM((2, EMBED * TPW, IL), jnp.float32),
            pltpu.SemaphoreType.DMA,
            pltpu.SemaphoreType.DMA,
            pltpu.SemaphoreType.DMA,
            pltpu.SemaphoreType.DMA,
        ],
        compiler_params=pltpu.CompilerParams(needs_layout_passes=False),
    )
    def k(tab_hbm, xq_hbm, out_hbm, tab_v, x_v, out_v, is0, is1, os0, os1):
        wid = lax.axis_index("s") * nc + lax.axis_index("c")
        a = wid // WB
        b = wid % WB
        ti0 = b * TPW
        pltpu.sync_copy(tab_hbm, tab_v)
        in_sems = (is0, is1)
        out_sems = (os0, os1)

        def in_copy(jj, p):
            j = a * JPW + jj
            jt = j // JS
            js = j - jt * JS
            return pltpu.make_async_copy(
                xq_hbm.at[jt, pl.ds(ti0, TPW), js, :], x_v.at[p], in_sems[p]
            )

        def out_copy(jj, p):
            j = a * JPW + jj
            return pltpu.make_async_copy(
                out_v.at[p],
                out_hbm.at[j, pl.ds(ti0 * EMBED, TPW * EMBED), :],
                out_sems[p],
            )

        def compute(p):
            @plsc.parallel_loop(0, TPW, unroll=6)
            def t_body(t):
                for v in range(IL // LANES):
                    xv = x_v[p, t, pl.ds(v * LANES, LANES)]
                    i4 = xv * EMBED
                    for c in range(EMBED):
                        vals = plsc.load_gather(tab_v, [i4 + c])
                        out_v[p, t * EMBED + c, pl.ds(v * LANES, LANES)] = vals

        in_copy(0, 0).start()

        def g_body(g, carry):
            for p in (0, 1):
                jj = g * 2 + p

                @pl.when(jj + 1 < JPW)
                def _():
                    in_copy(jj + 1, 1 - p).start()

                in_copy(jj, p).wait()

                @pl.when(jj >= 2)
                def _():
                    out_copy(jj - 2, p).wait()

                compute(p)
                out_copy(jj, p).start()
            return carry

        lax.fori_loop(0, JPW // 2, g_body, 0)
        out_copy(JPW - 2, 0).wait()
        out_copy(JPW - 1, 1).wait()

    return k


def kernel(x, table):
    xq = x.T.reshape(JT, JS, IT, IL).transpose(0, 2, 1, 3)
    tab_flat = jnp.pad(table.reshape(-1), (0, 32 - VOCAB * EMBED))
    po = _make_sc_kernel()(tab_flat, xq)
    out = po.reshape(NJ, IT, EMBED, IL).transpose(1, 3, 0, 2).reshape(NI, NJ, EMBED)
    return out


# speedup vs baseline: 1.0860x; 1.0860x over previous
import functools

import jax
import jax.numpy as jnp
from jax import lax
from jax.experimental import pallas as pl
from jax.experimental.pallas import tpu as pltpu
from jax.experimental.pallas import tpu_sc as plsc

VOCAB = 7
EMBED = 4
LANES = 16

NJ = 200
NI = 16384
JT, JS = 25, 8
IT, IL = 128, 128
WA, WB = 4, 8
JPW = NJ // WA
TPW = IT // WB


@functools.lru_cache(maxsize=None)
def _make_sc_kernel():
    mesh = plsc.VectorSubcoreMesh(core_axis_name="c", subcore_axis_name="s")
    info = plsc.get_sparse_core_info()
    nc = info.num_cores

    @functools.partial(
        pl.kernel,
        mesh=mesh,
        out_type=jax.ShapeDtypeStruct((NJ, EMBED * IT, IL), jnp.float32),
        scratch_types=[
            pltpu.VMEM((32,), jnp.float32),
            pltpu.VMEM((2, TPW, IL), jnp.int32),
            pltpu.VMEM((2, EMBED * TPW, IL), jnp.float32),
            pltpu.SemaphoreType.DMA,
            pltpu.SemaphoreType.DMA,
            pltpu.SemaphoreType.DMA,
            pltpu.SemaphoreType.DMA,
        ],
        compiler_params=pltpu.CompilerParams(needs_layout_passes=False),
    )
    def k(tab_hbm, xq_hbm, out_hbm, tab_v, x_v, out_v, is0, is1, os0, os1):
        wid = lax.axis_index("s") * nc + lax.axis_index("c")
        a = wid // WB
        b = wid % WB
        ti0 = b * TPW
        pltpu.sync_copy(tab_hbm, tab_v)
        in_sems = (is0, is1)
        out_sems = (os0, os1)

        def in_copy(jj, p):
            j = a * JPW + jj
            jt = j // JS
            js = j - jt * JS
            return pltpu.make_async_copy(
                xq_hbm.at[jt, pl.ds(ti0, TPW), js, :], x_v.at[p], in_sems[p]
            )

        def out_copy(jj, p):
            j = a * JPW + jj
            return pltpu.make_async_copy(
                out_v.at[p],
                out_hbm.at[j, pl.ds(ti0 * EMBED, TPW * EMBED), :],
                out_sems[p],
            )

        def compute(p):
            @plsc.parallel_loop(0, TPW, unroll=8)
            def t_body(t):
                for v in range(IL // LANES):
                    xv = x_v[p, t, pl.ds(v * LANES, LANES)]
                    i4 = xv * EMBED
                    for c in range(EMBED):
                        vals = plsc.load_gather(tab_v, [i4 + c])
                        out_v[p, t * EMBED + c, pl.ds(v * LANES, LANES)] = vals

        in_copy(0, 0).start()

        def g_body(g, carry):
            for p in (0, 1):
                jj = g * 2 + p

                @pl.when(jj + 1 < JPW)
                def _():
                    in_copy(jj + 1, 1 - p).start()

                in_copy(jj, p).wait()

                @pl.when(jj >= 2)
                def _():
                    out_copy(jj - 2, p).wait()

                compute(p)
                out_copy(jj, p).start()
            return carry

        lax.fori_loop(0, JPW // 2, g_body, 0)
        out_copy(JPW - 2, 0).wait()
        out_copy(JPW - 1, 1).wait()

    return k


def kernel(x, table):
    xq = x.T.reshape(JT, JS, IT, IL).transpose(0, 2, 1, 3)
    tab_flat = jnp.pad(table.reshape(-1), (0, 32 - VOCAB * EMBED))
    po = _make_sc_kernel()(tab_flat, xq)
    out = po.reshape(NJ, IT, EMBED, IL).transpose(1, 3, 0, 2).reshape(NI, NJ, EMBED)
    return out
